# Initial kernel scaffold; baseline (speedup 1.0000x reference)
import jax
import jax.numpy as jnp
from jax.experimental import pallas as pl


def kernel(edge_attr, W0, W1, W2, W3):
    raise NotImplementedError("write your pallas kernel here")


# speedup vs baseline: 6.2444x; 6.2444x over previous
import functools

import jax
import jax.numpy as jnp
from jax import lax
from jax.experimental import pallas as pl
from jax.experimental.pallas import tpu as pltpu
from jax.experimental.pallas import tpu_sc as plsc

E = 1_600_000
EMB = 32
D0, D1, D2, D3 = 6, 7, 3, 23
NROWS = D0 * D1 * D2 * D3

NW = 32
CHUNK = 512
NCHUNKS = E // CHUNK
BASE_PER_W = NCHUNKS // NW
EXTRA = NCHUNKS - BASE_PER_W * NW

_mesh = plsc.VectorSubcoreMesh(core_axis_name="c", subcore_axis_name="s")


@functools.partial(
    pl.kernel,
    mesh=_mesh,
    out_type=jax.ShapeDtypeStruct((E, EMB), jnp.float32),
    scratch_types=[
        pltpu.VMEM((4, CHUNK), jnp.int32),
        pltpu.VMEM((4, 128), jnp.int32),
        pltpu.VMEM((CHUNK, EMB), jnp.float32),
        pltpu.SemaphoreType.DMA,
    ],
    compiler_params=pltpu.CompilerParams(use_tc_tiling_on_sc=False),
)
def _lookup(eat_hbm, ct_hbm, out_hbm, col_v, cidx_v, rows_v, sem):
    cid = lax.axis_index("c")
    sid = lax.axis_index("s")
    wid = sid * 2 + cid
    n_chunks = BASE_PER_W + jnp.where(wid < EXTRA, 1, 0).astype(jnp.int32)

    def chunk_body(i, carry):
        chunk = wid + NW * i
        base = chunk * CHUNK
        for k in range(4):
            pltpu.sync_copy(eat_hbm.at[k, pl.ds(base, CHUNK)], col_v.at[k])
        dmas = []
        for j in range(4):
            for h in range(8):
                s = pl.ds((j * 8 + h) * 16, 16)
                c = (col_v[0, s] * (D1 * D2 * D3)
                     + col_v[1, s] * (D2 * D3)
                     + col_v[2, s] * D3
                     + col_v[3, s])
                cidx_v[j, pl.ds(h * 16, 16)] = c
            dmas.append(
                pltpu.async_copy(
                    ct_hbm.at[cidx_v.at[j]],
                    rows_v.at[pl.ds(j * 128, 128)],
                    sem,
                )
            )
        for d in dmas:
            d.wait()
        pltpu.sync_copy(rows_v, out_hbm.at[pl.ds(base, CHUNK)])
        return carry

    lax.fori_loop(0, n_chunks, chunk_body, 0)


def kernel(edge_attr, W0, W1, W2, W3):
    W0z = W0.at[0].set(0.0)
    W1z = W1.at[0].set(0.0)
    W2z = W2.at[0].set(0.0)
    W3z = W3.at[0].set(0.0)
    ct = (W0z[:, None, None, None, :]
          + W1z[None, :, None, None, :]
          + W2z[None, None, :, None, :]
          + W3z[None, None, None, :, :]).reshape(NROWS, EMB)
    eat = edge_attr.T
    return _lookup(eat, ct)
